# Initial kernel scaffold; baseline (speedup 1.0000x reference)
import jax
import jax.numpy as jnp
from jax.experimental import pallas as pl


def kernel(positions, weights):
    raise NotImplementedError("write your pallas kernel here")


# speedup vs baseline: 4.7352x; 4.7352x over previous
import functools

import jax
import jax.numpy as jnp
from jax import lax
from jax.experimental import pallas as pl
from jax.experimental.pallas import tpu as pltpu
from jax.experimental.pallas import tpu_sc as plsc

EMB = 64
IDXW = 128
K = 4
CH = K * IDXW


@functools.partial(jax.jit, static_argnames=("total_rows",))
def _gather_rows(idx2d, table, total_rows):
    info = plsc.get_sparse_core_info()
    nw = info.num_cores * info.num_subcores
    per_w = total_rows // nw
    n_chunks = per_w // CH
    mesh = plsc.VectorSubcoreMesh(core_axis_name="c", subcore_axis_name="s")

    @functools.partial(
        pl.kernel,
        mesh=mesh,
        out_type=jax.ShapeDtypeStruct((total_rows, EMB), jnp.float32),
        compiler_params=pltpu.CompilerParams(use_tc_tiling_on_sc=False),
        scratch_types=[
            pltpu.VMEM((K, IDXW), jnp.int32),
            pltpu.VMEM((CH, EMB), jnp.float32),
            pltpu.SemaphoreType.DMA,
        ],
    )
    def k(idx_hbm, table_hbm, out_hbm, idx_v, rows_v, sem):
        wid = lax.axis_index("s") * info.num_cores + lax.axis_index("c")
        row0 = wid * per_w
        idxrow0 = wid * (per_w // IDXW)

        def body(i, carry):
            base = row0 + i * CH
            pltpu.sync_copy(idx_hbm.at[pl.ds(idxrow0 + i * K, K), :], idx_v)
            handles = [
                pltpu.async_copy(
                    table_hbm.at[idx_v.at[j]],
                    rows_v.at[pl.ds(j * IDXW, IDXW), :],
                    sem,
                )
                for j in range(K)
            ]
            for h in handles:
                h.wait()
            pltpu.sync_copy(rows_v, out_hbm.at[pl.ds(base, CH), :])
            return carry

        lax.fori_loop(0, n_chunks, body, 0)

    return k(idx2d, table)


def kernel(positions, weights):
    total_rows = positions.shape[0] * positions.shape[1]
    idx2d = positions.reshape(total_rows // IDXW, IDXW).astype(jnp.int32)
    out = _gather_rows(idx2d, weights, total_rows)
    return lax.stop_gradient(out.reshape(positions.shape + (EMB,)))


# speedup vs baseline: 5.1220x; 1.0817x over previous
import functools

import jax
import jax.numpy as jnp
from jax import lax
from jax.experimental import pallas as pl
from jax.experimental.pallas import tpu as pltpu
from jax.experimental.pallas import tpu_sc as plsc

EMB = 64
IDXW = 128
K = 4
CH = K * IDXW
NBUF = 2


@functools.partial(jax.jit, static_argnames=("total_rows",))
def _gather_rows(idx2d, table, total_rows):
    info = plsc.get_sparse_core_info()
    nw = info.num_cores * info.num_subcores
    per_w = total_rows // nw
    n_chunks = per_w // CH
    n_outer = n_chunks // NBUF
    mesh = plsc.VectorSubcoreMesh(core_axis_name="c", subcore_axis_name="s")

    @functools.partial(
        pl.kernel,
        mesh=mesh,
        out_type=jax.ShapeDtypeStruct((total_rows, EMB), jnp.float32),
        compiler_params=pltpu.CompilerParams(use_tc_tiling_on_sc=False),
        scratch_types=[
            pltpu.VMEM((NBUF, K, IDXW), jnp.int32),
            pltpu.VMEM((NBUF, CH, EMB), jnp.float32),
            pltpu.SemaphoreType.DMA,
            pltpu.SemaphoreType.DMA,
            pltpu.SemaphoreType.DMA,
            pltpu.SemaphoreType.DMA,
            pltpu.SemaphoreType.DMA,
            pltpu.SemaphoreType.DMA,
        ],
    )
    def k(idx_hbm, table_hbm, out_hbm, idx_v, rows_v, si0, si1, sg0, sg1, ss0, ss1):
        wid = lax.axis_index("s") * info.num_cores + lax.axis_index("c")
        row0 = wid * per_w
        idxrow0 = wid * (per_w // IDXW)
        sem_i = (si0, si1)
        sem_g = (sg0, sg1)
        sem_s = (ss0, ss1)

        def fire_idx(ci, b):
            cj = jnp.minimum(ci, n_chunks - 1)
            pltpu.async_copy(
                idx_hbm.at[pl.ds(idxrow0 + cj * K, K), :], idx_v.at[b], sem_i[b]
            )

        def wait_idx(b):
            pltpu.make_async_copy(
                idx_hbm.at[pl.ds(0, K), :], idx_v.at[b], sem_i[b]
            ).wait()

        def gather_and_store(ci, b):
            wait_idx(b)
            handles = [
                pltpu.async_copy(
                    table_hbm.at[idx_v.at[b].at[j]],
                    rows_v.at[b].at[pl.ds(j * IDXW, IDXW), :],
                    sem_g[b],
                )
                for j in range(K)
            ]
            for h in handles:
                h.wait()
            fire_idx(ci + NBUF, b)
            pltpu.async_copy(rows_v.at[b], out_hbm.at[pl.ds(row0 + ci * CH, CH), :], sem_s[b])

        def wait_store(b):
            pltpu.make_async_copy(
                rows_v.at[b], out_hbm.at[pl.ds(row0, CH), :], sem_s[b]
            ).wait()

        for b in range(NBUF):
            fire_idx(b, b)
        for b in range(NBUF):
            gather_and_store(b, b)

        def body(g, carry):
            for b in range(NBUF):
                ci = g * NBUF + b
                wait_store(b)
                gather_and_store(ci, b)
            return carry

        lax.fori_loop(1, n_outer, body, 0)

        for b in range(NBUF):
            wait_store(b)
            wait_idx(b)

    return k(idx2d, table)


def kernel(positions, weights):
    total_rows = positions.shape[0] * positions.shape[1]
    idx2d = positions.reshape(total_rows // IDXW, IDXW).astype(jnp.int32)
    out = _gather_rows(idx2d, weights, total_rows)
    return lax.stop_gradient(out.reshape(positions.shape + (EMB,)))


# speedup vs baseline: 5.8208x; 1.1364x over previous
import functools

import jax
import jax.numpy as jnp
from jax import lax
from jax.experimental import pallas as pl
from jax.experimental.pallas import tpu as pltpu
from jax.experimental.pallas import tpu_sc as plsc

EMB = 64
IDXW = 128
K = 4
CH = K * IDXW
NBUF = 2


@functools.partial(jax.jit, static_argnames=("total_rows",))
def _gather_rows(idx2d, table, total_rows):
    info = plsc.get_sparse_core_info()
    nw = info.num_cores * info.num_subcores
    per_w = total_rows // nw
    n_chunks = per_w // CH
    n_outer = n_chunks // NBUF
    mesh = plsc.VectorSubcoreMesh(core_axis_name="c", subcore_axis_name="s")

    @functools.partial(
        pl.kernel,
        mesh=mesh,
        out_type=jax.ShapeDtypeStruct((total_rows, EMB), jnp.float32),
        compiler_params=pltpu.CompilerParams(use_tc_tiling_on_sc=False),
        scratch_types=[
            pltpu.VMEM((NBUF, K, IDXW), jnp.int32),
            pltpu.VMEM((NBUF, CH, EMB), jnp.float32),
            pltpu.VMEM_SHARED((8192, EMB), jnp.float32),
            pltpu.SemaphoreType.DMA,
            pltpu.SemaphoreType.DMA,
            pltpu.SemaphoreType.DMA,
            pltpu.SemaphoreType.DMA,
            pltpu.SemaphoreType.DMA,
            pltpu.SemaphoreType.DMA,
        ],
    )
    def k(idx_hbm, table_hbm, out_hbm, idx_v, rows_v, table_sp, si0, si1, sg0, sg1, ss0, ss1):
        wid = lax.axis_index("s") * info.num_cores + lax.axis_index("c")
        row0 = wid * per_w
        idxrow0 = wid * (per_w // IDXW)
        sem_i = (si0, si1)
        sem_g = (sg0, sg1)
        sem_s = (ss0, ss1)

        @pl.when(lax.axis_index("s") == 0)
        def _stage():
            pltpu.sync_copy(table_hbm, table_sp)

        plsc.subcore_barrier()

        def fire_idx(ci, b):
            cj = jnp.minimum(ci, n_chunks - 1)
            pltpu.async_copy(
                idx_hbm.at[pl.ds(idxrow0 + cj * K, K), :], idx_v.at[b], sem_i[b]
            )

        def wait_idx(b):
            pltpu.make_async_copy(
                idx_hbm.at[pl.ds(0, K), :], idx_v.at[b], sem_i[b]
            ).wait()

        def gather_and_store(ci, b):
            wait_idx(b)
            handles = [
                pltpu.async_copy(
                    table_sp.at[idx_v.at[b].at[j]],
                    rows_v.at[b].at[pl.ds(j * IDXW, IDXW), :],
                    sem_g[b],
                )
                for j in range(K)
            ]
            for h in handles:
                h.wait()
            fire_idx(ci + NBUF, b)
            pltpu.async_copy(rows_v.at[b], out_hbm.at[pl.ds(row0 + ci * CH, CH), :], sem_s[b])

        def wait_store(b):
            pltpu.make_async_copy(
                rows_v.at[b], out_hbm.at[pl.ds(row0, CH), :], sem_s[b]
            ).wait()

        for b in range(NBUF):
            fire_idx(b, b)
        for b in range(NBUF):
            gather_and_store(b, b)

        def body(g, carry):
            for b in range(NBUF):
                ci = g * NBUF + b
                wait_store(b)
                gather_and_store(ci, b)
            return carry

        lax.fori_loop(1, n_outer, body, 0)

        for b in range(NBUF):
            wait_store(b)
            wait_idx(b)

    return k(idx2d, table)


def kernel(positions, weights):
    total_rows = positions.shape[0] * positions.shape[1]
    idx2d = positions.reshape(total_rows // IDXW, IDXW).astype(jnp.int32)
    out = _gather_rows(idx2d, weights, total_rows)
    return lax.stop_gradient(out.reshape(positions.shape + (EMB,)))
